# Initial kernel scaffold; baseline (speedup 1.0000x reference)
import jax
import jax.numpy as jnp
from jax.experimental import pallas as pl


def kernel(x):
    raise NotImplementedError("write your pallas kernel here")


# speedup vs baseline: 3.0107x; 3.0107x over previous
import functools

import jax
import jax.numpy as jnp
from jax.experimental import pallas as pl

_W = 4
_K = 12
_NEG = float("-inf")


def _body(x_ref, o_ref):
    xv = x_ref[0]
    w = xv[:, 0:253] + xv[:, 1:254] + xv[:, 2:255] + xv[:, 3:256]
    acc = jnp.zeros((128, 1), jnp.float32)
    taken = jnp.zeros((128, 1), jnp.float32)
    for _ in range(_K):
        m = jnp.max(w, axis=1, keepdims=True)
        eq = w == m
        cnt = jnp.sum(eq.astype(jnp.float32), axis=1, keepdims=True)
        take = jnp.clip(float(_K) - taken, 0.0, cnt)
        acc = acc + jnp.where(take > 0, m * take, 0.0)
        taken = taken + take
        w = jnp.where(eq, _NEG, w)
    z = acc * (1.0 / (_K * _W))
    o_ref[0, 0, :] = jnp.full((128,), jnp.mean(z), jnp.float32)


@jax.jit
def kernel(x):
    B, C, F, T = x.shape
    xr = x.reshape(B * C, F, T)
    out = pl.pallas_call(
        _body,
        grid=(B * C,),
        in_specs=[pl.BlockSpec((1, F, T), lambda i: (i, 0, 0))],
        out_specs=pl.BlockSpec((1, 1, 128), lambda i: (i, 0, 0)),
        out_shape=jax.ShapeDtypeStruct((B * C, 1, 128), jnp.float32),
    )(xr)
    return out[:, 0, 0].reshape(B, C)


# speedup vs baseline: 5.0178x; 1.6666x over previous
import jax
import jax.numpy as jnp
from jax.experimental import pallas as pl

_W = 4
_K = 12
_TW = 253
_IMIN = -2147483648


def _body(x_ref, o_ref):
    xv = x_ref[0]
    w = xv
    w += jnp.concatenate([xv[:, 1:], xv[:, :1]], axis=1)
    w += jnp.concatenate([xv[:, 2:], xv[:, :2]], axis=1)
    w += jnp.concatenate([xv[:, 3:], xv[:, :3]], axis=1)
    bits = jax.lax.bitcast_convert_type(w, jnp.int32)
    key = jnp.where(bits < 0, bits ^ 0x7FFFFFFF, bits)
    t = jax.lax.broadcasted_iota(jnp.int32, (128, 256), 1)
    key = (key & -256) | t
    key = jnp.where(t < _TW, key, _IMIN)
    kt = key.T

    acc = jnp.zeros((1, 128), jnp.float32)
    for _ in range(_K):
        m = jnp.max(kt, axis=0, keepdims=True)
        kv = m & -256
        vbits = jnp.where(kv < 0, kv ^ 0x7FFFFFFF, kv)
        acc += jax.lax.bitcast_convert_type(vbits, jnp.float32)
        kt = jnp.where(kt == m, _IMIN, kt)
    z = jnp.mean(acc) * (1.0 / (_K * _W))
    o_ref[0, 0, :] = jnp.full((128,), z, jnp.float32)


@jax.jit
def kernel(x):
    B, C, F, T = x.shape
    xr = x.reshape(B * C, F, T)
    out = pl.pallas_call(
        _body,
        grid=(B * C,),
        in_specs=[pl.BlockSpec((1, F, T), lambda i: (i, 0, 0))],
        out_specs=pl.BlockSpec((1, 1, 128), lambda i: (i, 0, 0)),
        out_shape=jax.ShapeDtypeStruct((B * C, 1, 128), jnp.float32),
    )(xr)
    return out[:, 0, 0].reshape(B, C)


# speedup vs baseline: 8.2977x; 1.6537x over previous
import jax
import jax.numpy as jnp
from jax.experimental import pallas as pl

_W = 4
_K = 12
_TW = 253
_IMIN = -2147483648


_R = 256


def _body(x_ref, o_ref):
    xv = x_ref[...].reshape(_R, 256)
    w = xv
    w += jnp.concatenate([xv[:, 1:], xv[:, :1]], axis=1)
    w += jnp.concatenate([xv[:, 2:], xv[:, :2]], axis=1)
    w += jnp.concatenate([xv[:, 3:], xv[:, :3]], axis=1)
    bits = jax.lax.bitcast_convert_type(w, jnp.int32)
    key = jnp.where(bits < 0, bits ^ 0x7FFFFFFF, bits)
    t = jax.lax.broadcasted_iota(jnp.int32, (_R, 256), 1)
    key = (key & -256) | t
    key = jnp.where(t < _TW, key, _IMIN)
    kt = key.T

    acc = jnp.zeros((1, _R), jnp.float32)
    for _ in range(_K):
        m = jnp.max(kt, axis=0, keepdims=True)
        kv = m & -256
        vbits = jnp.where(kv < 0, kv ^ 0x7FFFFFFF, kv)
        acc += jax.lax.bitcast_convert_type(vbits, jnp.float32)
        kt = jnp.where(kt == m, _IMIN, kt)
    zz = jnp.mean(acc.reshape(_R // 128, 128), axis=1) * (1.0 / (_K * _W))
    o_ref[0] = jnp.broadcast_to(zz[:, None], (_R // 128, 128))


@jax.jit
def kernel(x):
    B, C, F, T = x.shape
    ntile = _R // F
    xr = x.reshape(B * C // ntile, ntile * F, T)
    out = pl.pallas_call(
        _body,
        grid=(B * C // ntile,),
        in_specs=[pl.BlockSpec((1, ntile * F, T), lambda i: (i, 0, 0))],
        out_specs=pl.BlockSpec((1, ntile, 128), lambda i: (i, 0, 0)),
        out_shape=jax.ShapeDtypeStruct((B * C // ntile, ntile, 128), jnp.float32),
    )(xr)
    return out[:, :, 0].reshape(B, C)


# speedup vs baseline: 9.8751x; 1.1901x over previous
import jax
import jax.numpy as jnp
from jax.experimental import pallas as pl

_W = 4
_K = 12
_TW = 253
_IMIN = -2147483648


_R = 512


def _body(x_ref, o_ref):
    xv = x_ref[...].reshape(_R, 256)
    w = xv
    w += jnp.concatenate([xv[:, 1:], xv[:, :1]], axis=1)
    w += jnp.concatenate([xv[:, 2:], xv[:, :2]], axis=1)
    w += jnp.concatenate([xv[:, 3:], xv[:, :3]], axis=1)
    bits = jax.lax.bitcast_convert_type(w, jnp.int32)
    key = jnp.where(bits < 0, bits ^ 0x7FFFFFFF, bits)
    t = jax.lax.broadcasted_iota(jnp.int32, (_R, 256), 1)
    key = (key & -256) | t
    key = jnp.where(t < _TW, key, _IMIN)
    kt = key.T

    acc = jnp.zeros((1, _R), jnp.float32)
    for _ in range(_K):
        m = jnp.max(kt, axis=0, keepdims=True)
        kv = m & -256
        vbits = jnp.where(kv < 0, kv ^ 0x7FFFFFFF, kv)
        acc += jax.lax.bitcast_convert_type(vbits, jnp.float32)
        kt = jnp.where(kt == m, _IMIN, kt)
    zz = jnp.mean(acc.reshape(_R // 128, 128), axis=1) * (1.0 / (_K * _W))
    o_ref[0] = jnp.broadcast_to(zz[:, None], (_R // 128, 128))


@jax.jit
def kernel(x):
    B, C, F, T = x.shape
    ntile = _R // F
    xr = x.reshape(B * C // ntile, ntile * F, T)
    out = pl.pallas_call(
        _body,
        grid=(B * C // ntile,),
        in_specs=[pl.BlockSpec((1, ntile * F, T), lambda i: (i, 0, 0))],
        out_specs=pl.BlockSpec((1, ntile, 128), lambda i: (i, 0, 0)),
        out_shape=jax.ShapeDtypeStruct((B * C // ntile, ntile, 128), jnp.float32),
    )(xr)
    return out[:, :, 0].reshape(B, C)


# speedup vs baseline: 10.6238x; 1.0758x over previous
import jax
import jax.numpy as jnp
from jax.experimental import pallas as pl

_W = 4
_K = 12
_TW = 253
_IMIN = -2147483648


_R = 1024


def _body(x_ref, o_ref):
    xv = x_ref[...].reshape(_R, 256)
    w = xv
    w += jnp.concatenate([xv[:, 1:], xv[:, :1]], axis=1)
    w += jnp.concatenate([xv[:, 2:], xv[:, :2]], axis=1)
    w += jnp.concatenate([xv[:, 3:], xv[:, :3]], axis=1)
    bits = jax.lax.bitcast_convert_type(w, jnp.int32)
    key = jnp.where(bits < 0, bits ^ 0x7FFFFFFF, bits)
    t = jax.lax.broadcasted_iota(jnp.int32, (_R, 256), 1)
    key = (key & -256) | t
    key = jnp.where(t < _TW, key, _IMIN)
    kt = key.T

    acc = jnp.zeros((1, _R), jnp.float32)
    for _ in range(_K):
        m = jnp.max(kt, axis=0, keepdims=True)
        kv = m & -256
        vbits = jnp.where(kv < 0, kv ^ 0x7FFFFFFF, kv)
        acc += jax.lax.bitcast_convert_type(vbits, jnp.float32)
        kt = jnp.where(kt == m, _IMIN, kt)
    zz = jnp.mean(acc.reshape(_R // 128, 128), axis=1) * (1.0 / (_K * _W))
    o_ref[0] = jnp.broadcast_to(zz[:, None], (_R // 128, 128))


@jax.jit
def kernel(x):
    B, C, F, T = x.shape
    ntile = _R // F
    xr = x.reshape(B * C // ntile, ntile * F, T)
    out = pl.pallas_call(
        _body,
        grid=(B * C // ntile,),
        in_specs=[pl.BlockSpec((1, ntile * F, T), lambda i: (i, 0, 0))],
        out_specs=pl.BlockSpec((1, ntile, 128), lambda i: (i, 0, 0)),
        out_shape=jax.ShapeDtypeStruct((B * C // ntile, ntile, 128), jnp.float32),
    )(xr)
    return out[:, :, 0].reshape(B, C)
